# baseline (speedup 1.0000x reference)
import functools
import jax
import jax.numpy as jnp
from jax import lax
from jax.experimental import pallas as pl
from jax.experimental.pallas import tpu as pltpu
from jax.experimental.pallas import tpu_sc as plsc

N = 10000
E = 320000
D = 128
CLS = 64
G = 64
PAD_N = 10240
NC = 2
NS = 16
NW = NC * NS
HALF = PAD_N // 2
ACC_R = HALF + 8
EB = 128
E0_PAD = 321536
ES0 = E0_PAD // 16
NB0 = ES0 // EB
EPT2 = E // 2
NB2 = EPT2 // EB
ROWS_PT = PAD_N // NW
PCHUNK = 64

_mesh = plsc.VectorSubcoreMesh(core_axis_name="c", subcore_axis_name="s")
f32 = jnp.float32
i32 = jnp.int32


def _deg_body(dst_hbm, deg_out, dbuf, dlbuf, cnt):
    cid = lax.axis_index("c")
    sid = lax.axis_index("s")
    wid = sid * NC + cid
    es = lax.rem(wid, 16)
    h = wid // 16
    lo = h * HALF

    def z(i, c):
        cnt[pl.ds(i * 16, 16)] = jnp.zeros((16,), f32)
        return c

    lax.fori_loop(0, ACC_R, z, 0)
    ones16 = jnp.ones((16,), f32)

    def blk(b, carry):
        pltpu.sync_copy(dst_hbm.at[pl.ds(es * ES0 + b * EB, EB)], dbuf)
        for g in range(EB // 16):
            dv = dbuf[pl.ds(g * 16, 16)]
            dl = dv - lo
            m = (dl >= 0) & (dl < HALF)
            dlbuf[pl.ds(g * 16, 16)] = jnp.where(m, dl, HALF)
        for g in range(EB // 16):
            dlv = dlbuf[pl.ds(g * 16, 16)]
            for lane in range(16):
                dls = dlv[lane]
                plsc.addupdate(cnt.at[pl.ds(dls * 16, 16)], ones16)
        return carry

    lax.fori_loop(0, NB0, blk, 0)
    pltpu.sync_copy(cnt.at[pl.ds(0, HALF * 16)],
                    deg_out.at[es, pl.ds(lo * 16, HALF * 16)])


_deg_call = pl.kernel(
    _deg_body,
    out_type=jax.ShapeDtypeStruct((16, PAD_N * 16), f32),
    mesh=_mesh,
    scratch_types=[
        pltpu.VMEM((EB,), i32),
        pltpu.VMEM((EB,), i32),
        pltpu.VMEM((ACC_R * 16,), f32),
    ],
)


def _scat_body(y_hbm, src_hbm, dst_hbm, acc_out, sbuf, dbuf, gbuf, dlbuf, rows, acc, sem):
    cid = lax.axis_index("c")
    sid = lax.axis_index("s")
    wid = sid * NC + cid
    e = wid & 1
    h = (wid >> 1) & 1
    f = wid >> 2
    lo = h * HALF
    e0 = e * EPT2

    def z(i, c):
        acc[pl.ds(i * 16, 16)] = jnp.zeros((16,), f32)
        return c

    lax.fori_loop(0, ACC_R, z, 0)

    def blk(b, carry):
        pltpu.sync_copy(src_hbm.at[pl.ds(e0 + b * EB, EB)], sbuf)
        pltpu.sync_copy(dst_hbm.at[pl.ds(e0 + b * EB, EB)], dbuf)
        for g in range(EB // 16):
            sv = sbuf[pl.ds(g * 16, 16)]
            gbuf[pl.ds(g * 16, 16)] = sv * 8 + f
            dv = dbuf[pl.ds(g * 16, 16)]
            dl = dv - lo
            m = (dl >= 0) & (dl < HALF)
            dlbuf[pl.ds(g * 16, 16)] = jnp.where(m, dl, HALF)
        pltpu.async_copy(y_hbm.at[gbuf], rows, sem).wait()
        for g in range(EB // 16):
            dlv = dlbuf[pl.ds(g * 16, 16)]
            for lane in range(16):
                dls = dlv[lane]
                plsc.addupdate(acc.at[pl.ds(dls * 16, 16)],
                               rows[g * 16 + lane, :])
        return carry

    lax.fori_loop(0, NB2, blk, 0)
    pltpu.sync_copy(acc.at[pl.ds(0, HALF * 16)], acc_out.at[e, h, f])


_scat_call = pl.kernel(
    _scat_body,
    out_type=jax.ShapeDtypeStruct((2, 2, 8, HALF * 16), f32),
    mesh=_mesh,
    scratch_types=[
        pltpu.VMEM((EB,), i32),
        pltpu.VMEM((EB,), i32),
        pltpu.VMEM((EB,), i32),
        pltpu.VMEM((EB,), i32),
        pltpu.VMEM((EB, 16), f32),
        pltpu.VMEM((ACC_R * 16,), f32),
        pltpu.SemaphoreType.DMA,
    ],
    compiler_params=pltpu.CompilerParams(use_tc_tiling_on_sc=False),
)


def _assemble(acc_ref):
    ap = acc_ref[...].reshape(2, 8, 1280, 16)
    aps = ap[0] + ap[1]
    return jnp.concatenate([aps[k] for k in range(8)], axis=1)


def _acc_spec():
    return pl.BlockSpec((2, 1, 8, 1280 * 16), lambda i: (0, i // 4, 0, i % 4))


def _l1_body(x_ref, w_ref, deg_ref, y_ref, dinv_ref):
    dp = deg_ref[...].reshape(16, 1280, 16)
    deg = dp.sum(axis=0)[:, :1]
    dinv = lax.rsqrt(deg + 1.0)
    xw = jnp.dot(x_ref[...], w_ref[...], preferred_element_type=f32)
    y_ref[...] = dinv * xw
    dinv_ref[...] = jnp.broadcast_to(dinv, dinv_ref.shape)


def _l1_call(xp, W1, degp):
    blk = PAD_N // 8
    return pl.pallas_call(
        _l1_body,
        grid=(8,),
        in_specs=[
            pl.BlockSpec((blk, D), lambda i: (i, 0)),
            pl.BlockSpec((D, D), lambda i: (0, 0)),
            pl.BlockSpec((16, blk * 16), lambda i: (0, i)),
        ],
        out_specs=[
            pl.BlockSpec((blk, D), lambda i: (i, 0)),
            pl.BlockSpec((blk, 16), lambda i: (i, 0)),
        ],
        out_shape=[
            jax.ShapeDtypeStruct((PAD_N, D), f32),
            jax.ShapeDtypeStruct((PAD_N, 16), f32),
        ],
    )(xp, W1, degp)


def _l2_body(acc_ref, y1_ref, dinv_ref, b1_ref, w2_ref, y2_ref):
    dinv = dinv_ref[:, :1]
    accb = _assemble(acc_ref)
    h1 = jnp.maximum(dinv * (accb + y1_ref[...]) + b1_ref[...], 0.0)
    y2_ref[...] = dinv * jnp.dot(h1, w2_ref[...], preferred_element_type=f32)


def _l2_call(accp, y1, dinvb, b1r, W2):
    blk = PAD_N // 8
    return pl.pallas_call(
        _l2_body,
        grid=(8,),
        in_specs=[
            _acc_spec(),
            pl.BlockSpec((blk, D), lambda i: (i, 0)),
            pl.BlockSpec((blk, 16), lambda i: (i, 0)),
            pl.BlockSpec((1, D), lambda i: (0, 0)),
            pl.BlockSpec((D, D), lambda i: (0, 0)),
        ],
        out_specs=pl.BlockSpec((blk, D), lambda i: (i, 0)),
        out_shape=jax.ShapeDtypeStruct((PAD_N, D), f32),
    )(accp, y1, dinvb, b1r, W2)


def _comb_body(acc_ref, out_ref):
    out_ref[...] = _assemble(acc_ref)


def _comb_call(accp):
    blk = PAD_N // 8
    return pl.pallas_call(
        _comb_body,
        grid=(8,),
        in_specs=[_acc_spec()],
        out_specs=pl.BlockSpec((blk, D), lambda i: (i, 0)),
        out_shape=jax.ShapeDtypeStruct((PAD_N, D), f32),
    )(accp)


def _pool_body(acc_hbm, y2_hbm, dinv_hbm, batch_hbm, b2_hbm,
               psum, pmax, pcnt,
               a0c, y2c, bat, dnv, b2v, lsum, lmax, lcnt):
    cid = lax.axis_index("c")
    sid = lax.axis_index("s")
    wid = sid * NC + cid
    n0 = wid * ROWS_PT
    valid = jnp.clip(N - n0, 0, ROWS_PT)
    pltpu.sync_copy(batch_hbm.at[pl.ds(n0 * 16, ROWS_PT * 16)], bat)
    pltpu.sync_copy(dinv_hbm.at[pl.ds(n0 * 16, ROWS_PT * 16)], dnv)
    pltpu.sync_copy(b2_hbm, b2v)

    def zrow(i, carry):
        z = jnp.zeros((16,), f32)
        lsum[pl.ds(i * 16, 16)] = z
        lmax[pl.ds(i * 16, 16)] = z
        return carry

    lax.fori_loop(0, G * D // 16, zrow, 0)

    def zc(i, carry):
        lcnt[pl.ds(i * 16, 16)] = jnp.zeros((16,), f32)
        return carry

    lax.fori_loop(0, G, zc, 0)

    for c in range(ROWS_PT // PCHUNK):
        r0 = n0 + c * PCHUNK
        pltpu.sync_copy(acc_hbm.at[pl.ds(r0 * D, PCHUNK * D)], a0c)
        pltpu.sync_copy(y2_hbm.at[pl.ds(r0 * D, PCHUNK * D)], y2c)
        nloc = jnp.clip(valid - c * PCHUNK, 0, PCHUNK)

        def node(j, carry, c=c):
            jg = c * PCHUNK + j
            g = bat[pl.ds(jg * 16, 16)][0]
            dv = dnv[pl.ds(jg * 16, 16)]
            base = g * D
            for f in range(D // 16):
                off = j * D + f * 16
                hh = dv * (a0c[pl.ds(off, 16)] + y2c[pl.ds(off, 16)])
                hh = jnp.maximum(hh + b2v[pl.ds(f * 16, 16)], 0.0)
                sl = pl.ds(base + f * 16, 16)
                lsum[sl] = lsum[sl] + hh
                lmax[sl] = jnp.maximum(lmax[sl], hh)
            cs = pl.ds(g * 16, 16)
            lcnt[cs] = lcnt[cs] + 1.0
            return carry

        lax.fori_loop(0, nloc, node, 0)

    pltpu.sync_copy(lsum, psum.at[wid])
    pltpu.sync_copy(lmax, pmax.at[wid])
    pltpu.sync_copy(lcnt, pcnt.at[wid])


_pool_call = pl.kernel(
    _pool_body,
    out_type=(
        jax.ShapeDtypeStruct((NW, G * D), f32),
        jax.ShapeDtypeStruct((NW, G * D), f32),
        jax.ShapeDtypeStruct((NW, G * 16), f32),
    ),
    mesh=_mesh,
    scratch_types=[
        pltpu.VMEM((PCHUNK * D,), f32),
        pltpu.VMEM((PCHUNK * D,), f32),
        pltpu.VMEM((ROWS_PT * 16,), i32),
        pltpu.VMEM((ROWS_PT * 16,), f32),
        pltpu.VMEM((D,), f32),
        pltpu.VMEM((G * D,), f32),
        pltpu.VMEM((G * D,), f32),
        pltpu.VMEM((G * 16,), f32),
    ],
)


def _head_body(psum_ref, pmax_ref, pcnt_ref, linW_ref, linb_ref, lin2W_ref, lin2b_ref, out_ref):
    ps = psum_ref[...].reshape(NW, G, D)
    pm = pmax_ref[...].reshape(NW, G, D)
    seg_sum = ps.sum(axis=0)
    seg_max = pm.max(axis=0)
    cnt = pcnt_ref[...].reshape(NW, G, 16).sum(axis=0)[:, :1]
    mean = seg_sum / jnp.maximum(cnt, 1.0)
    gfeat = jnp.concatenate([mean, seg_max, seg_sum], axis=1)
    z = jnp.maximum(jnp.dot(gfeat, linW_ref[...], preferred_element_type=f32) + linb_ref[...], 0.0)
    out_ref[...] = jax.nn.sigmoid(jnp.dot(z, lin2W_ref[...], preferred_element_type=f32) + lin2b_ref[...])


def _head_call(psum, pmax, pcnt, linW, linbr, lin2Wp, lin2bp):
    return pl.pallas_call(
        _head_body,
        out_shape=jax.ShapeDtypeStruct((G, 128), f32),
    )(psum, pmax, pcnt, linW, linbr, lin2Wp, lin2bp)


@jax.jit
def kernel(x, edge_index, pos, batch, W1, b1, W2, b2, linW, linb, lin2W, lin2b):
    del pos
    ei = edge_index.astype(i32)
    src = ei[0]
    dst = ei[1]
    dst0 = jnp.concatenate([dst, jnp.full((E0_PAD - E,), -1, i32)])
    xp = jnp.zeros((PAD_N, D), f32).at[:N].set(x)
    batchb = jnp.zeros((PAD_N, 16), i32).at[:N].set(
        jnp.broadcast_to(batch.astype(i32)[:, None], (N, 16)))

    degp = _deg_call(dst0)
    y1, dinvb = _l1_call(xp, W1, degp)
    accp1 = _scat_call(y1.reshape(PAD_N * 8, 16), src, dst)
    y2 = _l2_call(accp1, y1, dinvb, b1.reshape(1, D), W2)
    accp2 = _scat_call(y2.reshape(PAD_N * 8, 16), src, dst)
    acc2 = _comb_call(accp2)
    psum, pmax, pcnt = _pool_call(
        acc2.reshape(-1), y2.reshape(-1), dinvb.reshape(-1),
        batchb.reshape(-1), b2)
    lin2Wp = jnp.concatenate([lin2W, jnp.zeros((CLS, 127), f32)], axis=1)
    lin2bp = jnp.broadcast_to(lin2b.reshape(1, 1), (1, 128))
    out = _head_call(psum, pmax, pcnt, linW, linb.reshape(1, CLS), lin2Wp, lin2bp)
    return out[:, :1]


# speedup vs baseline: 2.1654x; 2.1654x over previous
import functools
import jax
import jax.numpy as jnp
from jax import lax
from jax.experimental import pallas as pl
from jax.experimental.pallas import tpu as pltpu
from jax.experimental.pallas import tpu_sc as plsc

N = 10000
E = 320000
D = 128
CLS = 64
G = 64
PAD_N = 10240
NC = 2
NS = 16
NW = NC * NS
HALF = PAD_N // 2
ACC_R = HALF + 8
EB = 128
E0_PAD = 321536
ES0 = E0_PAD // 16
NB0 = ES0 // EB
EPT2 = E // 2
NB2 = EPT2 // EB
ROWS_PT = PAD_N // NW
PCHUNK = 64

_mesh = plsc.VectorSubcoreMesh(core_axis_name="c", subcore_axis_name="s")
f32 = jnp.float32
i32 = jnp.int32


def _deg_body(dst_hbm, deg_out, dbuf, dlbuf, cnt):
    cid = lax.axis_index("c")
    sid = lax.axis_index("s")
    wid = sid * NC + cid
    es = lax.rem(wid, 16)
    h = wid // 16
    lo = h * HALF

    def z(i, c):
        cnt[pl.ds(i * 16, 16)] = jnp.zeros((16,), f32)
        return c

    lax.fori_loop(0, ACC_R, z, 0)
    ones16 = jnp.ones((16,), f32)

    def blk(b, carry):
        pltpu.sync_copy(dst_hbm.at[pl.ds(es * ES0 + b * EB, EB)], dbuf)
        for g in range(EB // 16):
            dv = dbuf[pl.ds(g * 16, 16)]
            dl = dv - lo
            m = (dl >= 0) & (dl < HALF)
            dlbuf[pl.ds(g * 16, 16)] = jnp.where(m, dl, HALF)
        for g in range(EB // 16):
            dlv = dlbuf[pl.ds(g * 16, 16)]
            for lane in range(16):
                dls = dlv[lane]
                plsc.addupdate(cnt.at[pl.ds(dls * 16, 16)], ones16)
        return carry

    lax.fori_loop(0, NB0, blk, 0)
    pltpu.sync_copy(cnt.at[pl.ds(0, HALF * 16)],
                    deg_out.at[es, pl.ds(lo * 16, HALF * 16)])


_deg_call = pl.kernel(
    _deg_body,
    out_type=jax.ShapeDtypeStruct((16, PAD_N * 16), f32),
    mesh=_mesh,
    scratch_types=[
        pltpu.VMEM((EB,), i32),
        pltpu.VMEM((EB,), i32),
        pltpu.VMEM((ACC_R * 16,), f32),
    ],
)


def _scat_body(y_hbm, src_hbm, dst_hbm, acc_out,
               sbuf2, dbuf2, gbuf2, dlbuf2, rows2, acc, isem, gsem):
    cid = lax.axis_index("c")
    sid = lax.axis_index("s")
    wid = sid * NC + cid
    e = wid & 1
    h = (wid >> 1) & 1
    f = wid >> 2
    lo = h * HALF
    e0 = e * EPT2

    def z(i, c):
        acc[pl.ds(i * 16, 16)] = jnp.zeros((16,), f32)
        return c

    lax.fori_loop(0, ACC_R, z, 0)

    def fire_idx(b, p):
        pltpu.async_copy(src_hbm.at[pl.ds(e0 + b * EB, EB)], sbuf2.at[p], isem[p])
        pltpu.async_copy(dst_hbm.at[pl.ds(e0 + b * EB, EB)], dbuf2.at[p], isem[p])

    def wait_idx(p):
        pltpu.make_async_copy(src_hbm.at[pl.ds(0, EB)], sbuf2.at[p], isem[p]).wait()
        pltpu.make_async_copy(src_hbm.at[pl.ds(0, EB)], dbuf2.at[p], isem[p]).wait()

    def transform(p):
        for g in range(EB // 16):
            sv = sbuf2[p, pl.ds(g * 16, 16)]
            gbuf2[p, pl.ds(g * 16, 16)] = sv * 8 + f
            dv = dbuf2[p, pl.ds(g * 16, 16)]
            dl = dv - lo
            m = (dl >= 0) & (dl < HALF)
            dlbuf2[p, pl.ds(g * 16, 16)] = jnp.where(m, dl, HALF)

    def fire_gather(p):
        pltpu.async_copy(y_hbm.at[gbuf2.at[p]], rows2.at[p], gsem[p])

    def wait_gather(p):
        pltpu.make_async_copy(y_hbm.at[gbuf2.at[p]], rows2.at[p], gsem[p]).wait()

    def adds(p):
        for g in range(EB // 16):
            dlv = dlbuf2[p, pl.ds(g * 16, 16)]
            for lane in range(16):
                dls = dlv[lane]
                plsc.addupdate(acc.at[pl.ds(dls * 16, 16)],
                               rows2[p, g * 16 + lane, :])

    fire_idx(0, 0)
    wait_idx(0)
    transform(0)
    fire_gather(0)
    fire_idx(1, 1)

    def blk(b, carry):
        for p in range(2):
            q = 1 - p

            @pl.when((b & 1) == p)
            def _(p=p, q=q):
                wait_gather(p)

                @pl.when(b + 1 < NB2)
                def _(p=p, q=q):
                    wait_idx(q)
                    transform(q)
                    fire_gather(q)

                    @pl.when(b + 2 < NB2)
                    def _(p=p):
                        fire_idx(b + 2, p)

                adds(p)
        return carry

    lax.fori_loop(0, NB2, blk, 0)
    pltpu.sync_copy(acc.at[pl.ds(0, HALF * 16)], acc_out.at[e, h, f])


_scat_call = pl.kernel(
    _scat_body,
    out_type=jax.ShapeDtypeStruct((2, 2, 8, HALF * 16), f32),
    mesh=_mesh,
    scratch_types=[
        pltpu.VMEM((2, EB), i32),
        pltpu.VMEM((2, EB), i32),
        pltpu.VMEM((2, EB), i32),
        pltpu.VMEM((2, EB), i32),
        pltpu.VMEM((2, EB, 16), f32),
        pltpu.VMEM((ACC_R * 16,), f32),
        [pltpu.SemaphoreType.DMA for _ in range(2)],
        [pltpu.SemaphoreType.DMA for _ in range(2)],
    ],
    compiler_params=pltpu.CompilerParams(use_tc_tiling_on_sc=False),
)


def _assemble(acc_ref):
    ap = acc_ref[...].reshape(2, 8, 1280, 16)
    aps = ap[0] + ap[1]
    return jnp.concatenate([aps[k] for k in range(8)], axis=1)


def _acc_spec():
    return pl.BlockSpec((2, 1, 8, 1280 * 16), lambda i: (0, i // 4, 0, i % 4))


def _l1_body(x_ref, w_ref, deg_ref, y_ref, dinv_ref):
    dp = deg_ref[...].reshape(16, 1280, 16)
    deg = dp.sum(axis=0)[:, :1]
    dinv = lax.rsqrt(deg + 1.0)
    xw = jnp.dot(x_ref[...], w_ref[...], preferred_element_type=f32)
    y_ref[...] = dinv * xw
    dinv_ref[...] = jnp.broadcast_to(dinv, dinv_ref.shape)


def _l1_call(xp, W1, degp):
    blk = PAD_N // 8
    return pl.pallas_call(
        _l1_body,
        grid=(8,),
        in_specs=[
            pl.BlockSpec((blk, D), lambda i: (i, 0)),
            pl.BlockSpec((D, D), lambda i: (0, 0)),
            pl.BlockSpec((16, blk * 16), lambda i: (0, i)),
        ],
        out_specs=[
            pl.BlockSpec((blk, D), lambda i: (i, 0)),
            pl.BlockSpec((blk, 16), lambda i: (i, 0)),
        ],
        out_shape=[
            jax.ShapeDtypeStruct((PAD_N, D), f32),
            jax.ShapeDtypeStruct((PAD_N, 16), f32),
        ],
    )(xp, W1, degp)


def _l2_body(acc_ref, y1_ref, dinv_ref, b1_ref, w2_ref, y2_ref):
    dinv = dinv_ref[:, :1]
    accb = _assemble(acc_ref)
    h1 = jnp.maximum(dinv * (accb + y1_ref[...]) + b1_ref[...], 0.0)
    y2_ref[...] = dinv * jnp.dot(h1, w2_ref[...], preferred_element_type=f32)


def _l2_call(accp, y1, dinvb, b1r, W2):
    blk = PAD_N // 8
    return pl.pallas_call(
        _l2_body,
        grid=(8,),
        in_specs=[
            _acc_spec(),
            pl.BlockSpec((blk, D), lambda i: (i, 0)),
            pl.BlockSpec((blk, 16), lambda i: (i, 0)),
            pl.BlockSpec((1, D), lambda i: (0, 0)),
            pl.BlockSpec((D, D), lambda i: (0, 0)),
        ],
        out_specs=pl.BlockSpec((blk, D), lambda i: (i, 0)),
        out_shape=jax.ShapeDtypeStruct((PAD_N, D), f32),
    )(accp, y1, dinvb, b1r, W2)


def _comb_body(acc_ref, out_ref):
    out_ref[...] = _assemble(acc_ref)


def _comb_call(accp):
    blk = PAD_N // 8
    return pl.pallas_call(
        _comb_body,
        grid=(8,),
        in_specs=[_acc_spec()],
        out_specs=pl.BlockSpec((blk, D), lambda i: (i, 0)),
        out_shape=jax.ShapeDtypeStruct((PAD_N, D), f32),
    )(accp)


def _pool_body(acc_hbm, y2_hbm, dinv_hbm, batch_hbm, b2_hbm,
               psum, pmax, pcnt,
               a0c, y2c, bat, dnv, b2v, lsum, lmax, lcnt):
    cid = lax.axis_index("c")
    sid = lax.axis_index("s")
    wid = sid * NC + cid
    n0 = wid * ROWS_PT
    valid = jnp.clip(N - n0, 0, ROWS_PT)
    pltpu.sync_copy(batch_hbm.at[pl.ds(n0 * 16, ROWS_PT * 16)], bat)
    pltpu.sync_copy(dinv_hbm.at[pl.ds(n0 * 16, ROWS_PT * 16)], dnv)
    pltpu.sync_copy(b2_hbm, b2v)

    def zrow(i, carry):
        z = jnp.zeros((16,), f32)
        lsum[pl.ds(i * 16, 16)] = z
        lmax[pl.ds(i * 16, 16)] = z
        return carry

    lax.fori_loop(0, G * D // 16, zrow, 0)

    def zc(i, carry):
        lcnt[pl.ds(i * 16, 16)] = jnp.zeros((16,), f32)
        return carry

    lax.fori_loop(0, G, zc, 0)

    for c in range(ROWS_PT // PCHUNK):
        r0 = n0 + c * PCHUNK
        pltpu.sync_copy(acc_hbm.at[pl.ds(r0 * D, PCHUNK * D)], a0c)
        pltpu.sync_copy(y2_hbm.at[pl.ds(r0 * D, PCHUNK * D)], y2c)
        nloc = jnp.clip(valid - c * PCHUNK, 0, PCHUNK)

        def node(j, carry, c=c):
            jg = c * PCHUNK + j
            g = bat[pl.ds(jg * 16, 16)][0]
            dv = dnv[pl.ds(jg * 16, 16)]
            base = g * D
            for f in range(D // 16):
                off = j * D + f * 16
                hh = dv * (a0c[pl.ds(off, 16)] + y2c[pl.ds(off, 16)])
                hh = jnp.maximum(hh + b2v[pl.ds(f * 16, 16)], 0.0)
                sl = pl.ds(base + f * 16, 16)
                lsum[sl] = lsum[sl] + hh
                lmax[sl] = jnp.maximum(lmax[sl], hh)
            cs = pl.ds(g * 16, 16)
            lcnt[cs] = lcnt[cs] + 1.0
            return carry

        lax.fori_loop(0, nloc, node, 0)

    pltpu.sync_copy(lsum, psum.at[wid])
    pltpu.sync_copy(lmax, pmax.at[wid])
    pltpu.sync_copy(lcnt, pcnt.at[wid])


_pool_call = pl.kernel(
    _pool_body,
    out_type=(
        jax.ShapeDtypeStruct((NW, G * D), f32),
        jax.ShapeDtypeStruct((NW, G * D), f32),
        jax.ShapeDtypeStruct((NW, G * 16), f32),
    ),
    mesh=_mesh,
    scratch_types=[
        pltpu.VMEM((PCHUNK * D,), f32),
        pltpu.VMEM((PCHUNK * D,), f32),
        pltpu.VMEM((ROWS_PT * 16,), i32),
        pltpu.VMEM((ROWS_PT * 16,), f32),
        pltpu.VMEM((D,), f32),
        pltpu.VMEM((G * D,), f32),
        pltpu.VMEM((G * D,), f32),
        pltpu.VMEM((G * 16,), f32),
    ],
)


def _head_body(psum_ref, pmax_ref, pcnt_ref, linW_ref, linb_ref, lin2W_ref, lin2b_ref, out_ref):
    ps = psum_ref[...].reshape(NW, G, D)
    pm = pmax_ref[...].reshape(NW, G, D)
    seg_sum = ps.sum(axis=0)
    seg_max = pm.max(axis=0)
    cnt = pcnt_ref[...].reshape(NW, G, 16).sum(axis=0)[:, :1]
    mean = seg_sum / jnp.maximum(cnt, 1.0)
    gfeat = jnp.concatenate([mean, seg_max, seg_sum], axis=1)
    z = jnp.maximum(jnp.dot(gfeat, linW_ref[...], preferred_element_type=f32) + linb_ref[...], 0.0)
    out_ref[...] = jax.nn.sigmoid(jnp.dot(z, lin2W_ref[...], preferred_element_type=f32) + lin2b_ref[...])


def _head_call(psum, pmax, pcnt, linW, linbr, lin2Wp, lin2bp):
    return pl.pallas_call(
        _head_body,
        out_shape=jax.ShapeDtypeStruct((G, 128), f32),
    )(psum, pmax, pcnt, linW, linbr, lin2Wp, lin2bp)


@jax.jit
def kernel(x, edge_index, pos, batch, W1, b1, W2, b2, linW, linb, lin2W, lin2b):
    del pos
    ei = edge_index.astype(i32)
    src = ei[0]
    dst = ei[1]
    dst0 = jnp.concatenate([dst, jnp.full((E0_PAD - E,), -1, i32)])
    xp = jnp.zeros((PAD_N, D), f32).at[:N].set(x)
    batchb = jnp.zeros((PAD_N, 16), i32).at[:N].set(
        jnp.broadcast_to(batch.astype(i32)[:, None], (N, 16)))

    degp = _deg_call(dst0)
    y1, dinvb = _l1_call(xp, W1, degp)
    accp1 = _scat_call(y1.reshape(PAD_N * 8, 16), src, dst)
    y2 = _l2_call(accp1, y1, dinvb, b1.reshape(1, D), W2)
    accp2 = _scat_call(y2.reshape(PAD_N * 8, 16), src, dst)
    acc2 = _comb_call(accp2)
    psum, pmax, pcnt = _pool_call(
        acc2.reshape(-1), y2.reshape(-1), dinvb.reshape(-1),
        batchb.reshape(-1), b2)
    lin2Wp = jnp.concatenate([lin2W, jnp.zeros((CLS, 127), f32)], axis=1)
    lin2bp = jnp.broadcast_to(lin2b.reshape(1, 1), (1, 128))
    out = _head_call(psum, pmax, pcnt, linW, linb.reshape(1, CLS), lin2Wp, lin2bp)
    return out[:, :1]
